# Initial kernel scaffold; baseline (speedup 1.0000x reference)
import jax
import jax.numpy as jnp
from jax.experimental import pallas as pl


def kernel(z, codebook):
    raise NotImplementedError("write your pallas kernel here")


# speedup vs baseline: 1.4455x; 1.4455x over previous
import functools

import jax
import jax.numpy as jnp
from jax import lax
from jax.experimental import pallas as pl
from jax.experimental.pallas import tpu as pltpu
from jax.experimental.pallas import tpu_sc as plsc

N_EMB = 8192
D = 32
M_BLOCK = 512


def _vq_argmin_body(z2_ref, cb_ref, zn_ref, cbn_ref, idx_ref, loss_ref):
    ab2 = jnp.dot(
        z2_ref[...] * -2.0, cb_ref[...], preferred_element_type=jnp.float32
    )
    scores = (zn_ref[...] + ab2) + cbn_ref[...]
    rowmin = jnp.min(scores, axis=1)
    idx_ref[...] = jnp.argmin(scores, axis=1).astype(jnp.int32)
    part = jnp.sum(rowmin)

    @pl.when(pl.program_id(0) == 0)
    def _init():
        loss_ref[0, 0] = 0.0

    loss_ref[0, 0] += part


@jax.jit
def _vq_argmin(flat, codebook):
    m = flat.shape[0]
    grid = m // M_BLOCK
    zn = jnp.sum(flat * flat, axis=1)[:, None]
    cbn = jnp.sum(codebook * codebook, axis=1)[None, :]
    idx, loss_sum = pl.pallas_call(
        _vq_argmin_body,
        grid=(grid,),
        in_specs=[
            pl.BlockSpec((M_BLOCK, D), lambda i: (i, 0)),
            pl.BlockSpec((D, N_EMB), lambda i: (0, 0)),
            pl.BlockSpec((M_BLOCK, 1), lambda i: (i, 0)),
            pl.BlockSpec((1, N_EMB), lambda i: (0, 0)),
        ],
        out_specs=[
            pl.BlockSpec((M_BLOCK,), lambda i: (i,)),
            pl.BlockSpec((1, 1), lambda i: (0, 0), memory_space=pltpu.SMEM),
        ],
        out_shape=[
            jax.ShapeDtypeStruct((m,), jnp.int32),
            jax.ShapeDtypeStruct((1, 1), jnp.float32),
        ],
    )(flat, codebook.T, zn, cbn)
    return idx, loss_sum


def _make_sc_gather(m):
    info = plsc.get_sparse_core_info()
    nw = info.num_cores * info.num_subcores
    b_per_w = m // nw
    mesh = plsc.VectorSubcoreMesh(core_axis_name="c", subcore_axis_name="s")

    @functools.partial(
        pl.kernel,
        mesh=mesh,
        out_type=jax.ShapeDtypeStruct((m, D), jnp.float32),
        scratch_types=[
            pltpu.VMEM((b_per_w,), jnp.int32),
            pltpu.VMEM((b_per_w, D), jnp.float32),
            pltpu.SemaphoreType.DMA,
        ],
        compiler_params=pltpu.CompilerParams(use_tc_tiling_on_sc=False),
    )
    def gather_kernel(table_hbm, idx_hbm, out_hbm, idx_v, rows_v, sem):
        wid = lax.axis_index("s") * info.num_cores + lax.axis_index("c")
        base = wid * b_per_w
        pltpu.sync_copy(idx_hbm.at[pl.ds(base, b_per_w)], idx_v)
        copies = [
            pltpu.async_copy(
                table_hbm.at[idx_v.at[pl.ds(c * 128, 128)]],
                rows_v.at[pl.ds(c * 128, 128), :],
                sem,
            )
            for c in range(b_per_w // 128)
        ]
        for cp in copies:
            cp.wait()
        pltpu.sync_copy(rows_v, out_hbm.at[pl.ds(base, b_per_w)])

    return gather_kernel


@jax.jit
def _vq_full(flat, codebook):
    idx, loss_sum = _vq_argmin(flat, codebook)
    q = _make_sc_gather(flat.shape[0])(codebook, idx)
    return q, loss_sum


def kernel(z, codebook):
    B, T, d = z.shape
    flat = z.reshape(-1, d)
    q, loss_sum = _vq_full(flat, codebook)
    loss = 1.25 * loss_sum[0, 0] / (flat.shape[0] * d)
    return q.reshape(B, T, d), loss
